# baseline (speedup 1.0000x reference)
import functools

import jax
import jax.numpy as jnp
from jax import lax
from jax.experimental import pallas as pl
from jax.experimental.pallas import tpu as pltpu
from jax.experimental.pallas import tpu_sc as plsc

N = 10000
D = 128
E = 320000

NC = 2
NS = 16
NW = NC * NS

CHUNK = 128
CPW = 80
E_PAD = NW * CPW * CHUNK
NROWS = E_PAD // CHUNK
ZR = 632
N_ACC = NS * ZR

_mesh = plsc.VectorSubcoreMesh(core_axis_name="c", subcore_axis_name="s")


N_HR = 128


@functools.partial(
    pl.kernel,
    out_type=jax.ShapeDtypeStruct((NC, N_HR, 128), jnp.float32),
    mesh=_mesh,
    scratch_types=[
        pltpu.VMEM((CPW, CHUNK), jnp.int32),
        pltpu.VMEM((N_HR, 128), jnp.float32),
        pltpu.VMEM((N_HR,), jnp.int32),
        pltpu.VMEM_SHARED((N_HR, 128), jnp.float32),
    ],
    compiler_params=pltpu.CompilerParams(needs_layout_passes=False),
)
def _deg_kernel(dst2_hbm, out_hbm, idx_v, hist_v, id_v, hist_sh):
    cid = lax.axis_index("c")
    sid = lax.axis_index("s")
    wid = sid * NC + cid
    pltpu.sync_copy(dst2_hbm.at[pl.ds(wid * CPW, CPW)], idx_v)

    zeros16 = jnp.zeros((16,), jnp.float32)
    ones16 = jnp.ones((16,), jnp.float32)

    @pl.loop(0, N_HR)
    def _(i):
        for j in range(8):
            hist_v[i, pl.ds(j * 16, 16)] = zeros16

    @pl.loop(0, N_HR // 16)
    def _(k):
        id_v[pl.ds(k * 16, 16)] = lax.iota(jnp.int32, 16) + k * 16

    @pl.when(sid == 0)
    def _():
        pltpu.sync_copy(hist_v, hist_sh)

    plsc.subcore_barrier()

    @pl.loop(0, CPW * CHUNK // 16)
    def _(e):
        r = lax.shift_right_logical(e, 3)
        c = jnp.bitwise_and(e, 7) * 16
        v = idx_v[r, pl.ds(c, 16)]
        plsc.addupdate_scatter(
            hist_v,
            [lax.shift_right_logical(v, 7), jnp.bitwise_and(v, 127)],
            ones16,
        )

    pltpu.sync_copy(hist_v, hist_sh.at[id_v], add=True)
    plsc.subcore_barrier()
    pltpu.sync_copy(
        hist_sh.at[pl.ds(sid * 8, 8)], out_hbm.at[cid, pl.ds(sid * 8, 8)]
    )


@functools.partial(
    pl.kernel,
    out_type=jax.ShapeDtypeStruct((NC, N_ACC, D), jnp.float32),
    mesh=_mesh,
    scratch_types=[
        pltpu.VMEM((CPW, CHUNK), jnp.int32),
        pltpu.VMEM((CHUNK,), jnp.int32),
        pltpu.VMEM((CHUNK,), jnp.int32),
        pltpu.VMEM((CHUNK, D), jnp.float32),
        pltpu.VMEM((CHUNK, D), jnp.float32),
        pltpu.VMEM_SHARED((N_ACC, D), jnp.float32),
        pltpu.SemaphoreType.DMA,
        pltpu.SemaphoreType.DMA,
        pltpu.SemaphoreType.DMA,
        pltpu.SemaphoreType.DMA,
    ],
)
def _agg_kernel(src2_hbm, dst2_hbm, g_hbm, zeros_hbm, out_hbm,
                src_v, dst_a, dst_b, rows_a, rows_b, acc_sh,
                sem_a, sem_b, sem_sa, sem_sb):
    cid = lax.axis_index("c")
    sid = lax.axis_index("s")
    wid = sid * NC + cid
    base = wid * CPW
    pltpu.sync_copy(src2_hbm.at[pl.ds(base, CPW)], src_v)
    pltpu.sync_copy(zeros_hbm, acc_sh.at[pl.ds(sid * ZR, ZR)])
    plsc.subcore_barrier()

    def gather(i, rows, sem):
        return pltpu.make_async_copy(g_hbm.at[src_v.at[i]], rows, sem)

    def load_dst(i, dst_buf):
        pltpu.sync_copy(dst2_hbm.at[base + i], dst_buf)

    def scatter(rows, dst_buf, sem):
        return pltpu.make_async_copy(rows, acc_sh.at[dst_buf], sem)

    load_dst(0, dst_a)
    gather(0, rows_a, sem_a).start()
    load_dst(1, dst_b)
    gather(1, rows_b, sem_b).start()

    @pl.loop(0, CPW // 2)
    def _(k):
        i = k * 2
        gather(i, rows_a, sem_a).wait()
        pltpu.async_copy(rows_a, acc_sh.at[dst_a], sem_sa, add=True)
        gather(i + 1, rows_b, sem_b).wait()
        pltpu.async_copy(rows_b, acc_sh.at[dst_b], sem_sb, add=True)

        @pl.when(k < CPW // 2 - 1)
        def _():
            scatter(rows_a, dst_a, sem_sa).wait()
            load_dst(i + 2, dst_a)
            gather(i + 2, rows_a, sem_a).start()
            scatter(rows_b, dst_b, sem_sb).wait()
            load_dst(i + 3, dst_b)
            gather(i + 3, rows_b, sem_b).start()

        @pl.when(k == CPW // 2 - 1)
        def _():
            scatter(rows_a, dst_a, sem_sa).wait()
            scatter(rows_b, dst_b, sem_sb).wait()

    plsc.subcore_barrier()
    pltpu.sync_copy(
        acc_sh.at[pl.ds(sid * ZR, ZR)], out_hbm.at[cid, pl.ds(sid * ZR, ZR)]
    )


def _scale_body(x_ref, w_ref, d0_ref, d1_ref, g_ref, dinv_ref):
    h = jnp.dot(x_ref[...], w_ref[...], preferred_element_type=jnp.float32)
    dinv = lax.rsqrt(d0_ref[...] + d1_ref[...] + 1.0)
    dinv_ref[...] = dinv
    g_ref[...] = h * dinv


def _epilogue_body(dinv_ref, g_ref, acc_ref, b_ref, o_ref):
    o_ref[...] = (
        dinv_ref[...] * (acc_ref[0] + acc_ref[1] + g_ref[...]) + b_ref[...]
    )


_BR = 2000


def kernel(x, edge_index, W, b):
    src = edge_index[0].astype(jnp.int32)
    dst = edge_index[1].astype(jnp.int32)
    pad = E_PAD - E
    pad_i = jnp.arange(pad, dtype=jnp.int32)
    src_p = jnp.concatenate([src, pad_i % N])
    dst_p = jnp.concatenate([dst, N + (pad_i % 16)])
    src2 = src_p.reshape(NROWS, CHUNK)
    dst2 = dst_p.reshape(NROWS, CHUNK)

    zerosD = jnp.zeros((ZR, D), jnp.float32)

    deg = _deg_kernel(dst2).reshape(NC, N_HR * 128)
    d0 = deg[0, :N, None]
    d1 = deg[1, :N, None]

    grid = (N // _BR,)
    row_spec = pl.BlockSpec((_BR, D), lambda i: (i, 0))
    col_spec = pl.BlockSpec((_BR, 1), lambda i: (i, 0))
    g, dinv = pl.pallas_call(
        _scale_body,
        grid=grid,
        in_specs=[
            row_spec,
            pl.BlockSpec((D, D), lambda i: (0, 0)),
            col_spec,
            col_spec,
        ],
        out_specs=[row_spec, col_spec],
        out_shape=[
            jax.ShapeDtypeStruct((N, D), jnp.float32),
            jax.ShapeDtypeStruct((N, 1), jnp.float32),
        ],
    )(x, W, d0, d1)

    acc = _agg_kernel(src2, dst2, g, zerosD)

    out = pl.pallas_call(
        _epilogue_body,
        grid=grid,
        in_specs=[
            col_spec,
            row_spec,
            pl.BlockSpec((NC, _BR, D), lambda i: (0, i, 0)),
            pl.BlockSpec((1, D), lambda i: (0, 0)),
        ],
        out_specs=row_spec,
        out_shape=jax.ShapeDtypeStruct((N, D), jnp.float32),
    )(dinv, g, acc, b.reshape(1, D))
    return out


# speedup vs baseline: 1.0819x; 1.0819x over previous
import functools

import jax
import jax.numpy as jnp
from jax import lax
from jax.experimental import pallas as pl
from jax.experimental.pallas import tpu as pltpu
from jax.experimental.pallas import tpu_sc as plsc

N = 10000
D = 128
E = 320000

NC = 2
NS = 16
NW = NC * NS

CHUNK = 128
CPW = 80
E_PAD = NW * CPW * CHUNK
NROWS = E_PAD // CHUNK
ZR = 632
N_ACC = NS * ZR

_mesh = plsc.VectorSubcoreMesh(core_axis_name="c", subcore_axis_name="s")


N_HR = 128


@functools.partial(
    pl.kernel,
    out_type=jax.ShapeDtypeStruct((NC, N_HR, 128), jnp.float32),
    mesh=_mesh,
    scratch_types=[
        pltpu.VMEM((CPW, CHUNK), jnp.int32),
        pltpu.VMEM((N_HR, 128), jnp.float32),
        pltpu.VMEM((N_HR,), jnp.int32),
        pltpu.VMEM_SHARED((N_HR, 128), jnp.float32),
    ],
    compiler_params=pltpu.CompilerParams(needs_layout_passes=False),
)
def _deg_kernel(dst2_hbm, out_hbm, idx_v, hist_v, id_v, hist_sh):
    cid = lax.axis_index("c")
    sid = lax.axis_index("s")
    wid = sid * NC + cid
    pltpu.sync_copy(dst2_hbm.at[pl.ds(wid * CPW, CPW)], idx_v)

    zeros16 = jnp.zeros((16,), jnp.float32)
    ones16 = jnp.ones((16,), jnp.float32)

    @pl.loop(0, N_HR)
    def _(i):
        for j in range(8):
            hist_v[i, pl.ds(j * 16, 16)] = zeros16

    @pl.loop(0, N_HR // 16)
    def _(k):
        id_v[pl.ds(k * 16, 16)] = lax.iota(jnp.int32, 16) + k * 16

    @pl.when(sid == 0)
    def _():
        pltpu.sync_copy(hist_v, hist_sh)

    plsc.subcore_barrier()

    @pl.loop(0, CPW * CHUNK // 16)
    def _(e):
        r = lax.shift_right_logical(e, 3)
        c = jnp.bitwise_and(e, 7) * 16
        v = idx_v[r, pl.ds(c, 16)]
        plsc.addupdate_scatter(
            hist_v,
            [lax.shift_right_logical(v, 7), jnp.bitwise_and(v, 127)],
            ones16,
        )

    pltpu.sync_copy(hist_v, hist_sh.at[id_v], add=True)
    plsc.subcore_barrier()
    pltpu.sync_copy(
        hist_sh.at[pl.ds(sid * 8, 8)], out_hbm.at[cid, pl.ds(sid * 8, 8)]
    )


@functools.partial(
    pl.kernel,
    out_type=jax.ShapeDtypeStruct((NC, N_ACC, D), jnp.float32),
    mesh=_mesh,
    scratch_types=[
        pltpu.VMEM((CPW, CHUNK), jnp.int32),
        pltpu.VMEM((CHUNK,), jnp.int32),
        pltpu.VMEM((CHUNK,), jnp.int32),
        pltpu.VMEM((CHUNK, D), jnp.float32),
        pltpu.VMEM((CHUNK, D), jnp.float32),
        pltpu.VMEM_SHARED((N_ACC, D), jnp.float32),
        pltpu.SemaphoreType.DMA,
        pltpu.SemaphoreType.DMA,
    ],
)
def _agg_kernel(src2_hbm, dst2_hbm, g_hbm, zeros_hbm, out_hbm,
                src_v, dst_a, dst_b, rows_a, rows_b, acc_sh, sem_a, sem_b):
    cid = lax.axis_index("c")
    sid = lax.axis_index("s")
    wid = sid * NC + cid
    base = wid * CPW
    pltpu.sync_copy(src2_hbm.at[pl.ds(base, CPW)], src_v)
    pltpu.sync_copy(zeros_hbm, acc_sh.at[pl.ds(sid * ZR, ZR)])
    plsc.subcore_barrier()

    def gather(i, rows, sem):
        return pltpu.make_async_copy(g_hbm.at[src_v.at[i]], rows, sem)

    def load_dst(i, dst_buf):
        pltpu.sync_copy(dst2_hbm.at[base + i], dst_buf)

    load_dst(0, dst_a)
    gather(0, rows_a, sem_a).start()

    @pl.loop(0, CPW // 2)
    def _(k):
        i = k * 2
        load_dst(i + 1, dst_b)
        gather(i + 1, rows_b, sem_b).start()
        gather(i, rows_a, sem_a).wait()
        pltpu.sync_copy(rows_a, acc_sh.at[dst_a], add=True)

        @pl.when(k < CPW // 2 - 1)
        def _():
            load_dst(i + 2, dst_a)
            gather(i + 2, rows_a, sem_a).start()

        gather(i + 1, rows_b, sem_b).wait()
        pltpu.sync_copy(rows_b, acc_sh.at[dst_b], add=True)

    plsc.subcore_barrier()
    pltpu.sync_copy(
        acc_sh.at[pl.ds(sid * ZR, ZR)], out_hbm.at[cid, pl.ds(sid * ZR, ZR)]
    )


def _scale_body(x_ref, w_ref, d0_ref, d1_ref, g_ref, dinv_ref):
    h = jnp.dot(x_ref[...], w_ref[...], preferred_element_type=jnp.float32)
    dinv = lax.rsqrt(d0_ref[...] + d1_ref[...] + 1.0)
    dinv_ref[...] = dinv
    g_ref[...] = h * dinv


def _epilogue_body(dinv_ref, g_ref, acc_ref, b_ref, o_ref):
    o_ref[...] = (
        dinv_ref[...] * (acc_ref[0] + acc_ref[1] + g_ref[...]) + b_ref[...]
    )


_BR = 2000


def kernel(x, edge_index, W, b):
    src = edge_index[0].astype(jnp.int32)
    dst = edge_index[1].astype(jnp.int32)
    pad = E_PAD - E
    pad_i = jnp.arange(pad, dtype=jnp.int32)
    src_p = jnp.concatenate([src, pad_i % N])
    dst_p = jnp.concatenate([dst, N + (pad_i % 16)])
    src2 = src_p.reshape(NROWS, CHUNK)
    dst2 = dst_p.reshape(NROWS, CHUNK)

    zerosD = jnp.zeros((ZR, D), jnp.float32)

    deg = _deg_kernel(dst2).reshape(NC, N_HR * 128)
    d0 = deg[0, :N, None]
    d1 = deg[1, :N, None]

    grid = (N // _BR,)
    row_spec = pl.BlockSpec((_BR, D), lambda i: (i, 0))
    col_spec = pl.BlockSpec((_BR, 1), lambda i: (i, 0))
    g, dinv = pl.pallas_call(
        _scale_body,
        grid=grid,
        in_specs=[
            row_spec,
            pl.BlockSpec((D, D), lambda i: (0, 0)),
            col_spec,
            col_spec,
        ],
        out_specs=[row_spec, col_spec],
        out_shape=[
            jax.ShapeDtypeStruct((N, D), jnp.float32),
            jax.ShapeDtypeStruct((N, 1), jnp.float32),
        ],
    )(x, W, d0, d1)

    acc = _agg_kernel(src2, dst2, g, zerosD)

    out = pl.pallas_call(
        _epilogue_body,
        grid=grid,
        in_specs=[
            col_spec,
            row_spec,
            pl.BlockSpec((NC, _BR, D), lambda i: (0, i, 0)),
            pl.BlockSpec((1, D), lambda i: (0, 0)),
        ],
        out_specs=row_spec,
        out_shape=jax.ShapeDtypeStruct((N, D), jnp.float32),
    )(dinv, g, acc, b.reshape(1, D))
    return out
